# baseline (speedup 1.0000x reference)
import functools

import jax
import jax.numpy as jnp
from jax import lax
from jax.experimental import pallas as pl
from jax.experimental.pallas import tpu as pltpu
from jax.experimental.pallas import tpu_sc as plsc

VOCAB = 65024
DIM = 16
ROWS = 16384
COLS = 200
B = ROWS * COLS

NC, NS = 2, 16
NW = NC * NS
B_PER_W = B // NW
CHUNK = 800
NCHUNK = B_PER_W // CHUNK
NBUF = 4
NROUND = NCHUNK // NBUF
STAGE_ROWS = VOCAB // NS
SUBSTAGE = 508
NSTAGE = STAGE_ROWS // SUBSTAGE

_mesh = plsc.VectorSubcoreMesh(core_axis_name="c", subcore_axis_name="s")


@functools.partial(
    pl.kernel,
    out_type=jax.ShapeDtypeStruct((B, DIM), jnp.float32),
    mesh=_mesh,
    scratch_types=[pltpu.VMEM((CHUNK,), jnp.int32)] * NBUF
    + [pltpu.VMEM((CHUNK, DIM), jnp.float32)] * NBUF
    + [pltpu.SemaphoreType.DMA] * (3 * NBUF)
    + [
        pltpu.VMEM_SHARED((VOCAB, DIM), jnp.float32),
        pltpu.VMEM((SUBSTAGE, DIM), jnp.float32),
    ],
    compiler_params=pltpu.CompilerParams(use_tc_tiling_on_sc=False),
)
def _gather_kernel(x_hbm, table_hbm, out_hbm, *scratch):
    idx_v = scratch[0:NBUF]
    rows_v = scratch[NBUF : 2 * NBUF]
    sems = scratch[2 * NBUF : 5 * NBUF]
    sem_i = sems[0:NBUF]
    sem_g = sems[NBUF : 2 * NBUF]
    sem_o = sems[2 * NBUF : 3 * NBUF]
    tab_sh = scratch[5 * NBUF]
    stage_v = scratch[5 * NBUF + 1]

    sid = lax.axis_index("s")
    wid = sid * NC + lax.axis_index("c")
    base = wid * B_PER_W

    row0 = sid * STAGE_ROWS
    for k in range(NSTAGE):
        r0 = row0 + k * SUBSTAGE
        pltpu.sync_copy(table_hbm.at[pl.ds(r0, SUBSTAGE)], stage_v)
        pltpu.sync_copy(stage_v, tab_sh.at[pl.ds(r0, SUBSTAGE)])
    plsc.subcore_barrier()

    def idx_cp(g, b):
        return pltpu.make_async_copy(
            x_hbm.at[pl.ds(base + g * CHUNK, CHUNK)], idx_v[b], sem_i[b]
        )

    def gather_cp(b):
        src = tab_sh if b % 2 == 0 else table_hbm
        return pltpu.make_async_copy(
            src.at[idx_v[b]], rows_v[b], sem_g[b]
        )

    def store_cp(g, b):
        return pltpu.make_async_copy(
            rows_v[b], out_hbm.at[pl.ds(base + g * CHUNK, CHUNK)], sem_o[b]
        )

    for b in range(NBUF):
        idx_cp(b, b).start()
    for g in range(NBUF):
        idx_cp(g, g).wait()
        gather_cp(g).start()
        if g > 0:
            gather_cp(g - 1).wait()
            store_cp(g - 1, g - 1).start()
            idx_cp(g - 1 + NBUF, g - 1).start()

    def round_body(r, carry):
        g0 = r * NBUF
        for b in range(NBUF):
            g = g0 + b
            bp = (b - 1) % NBUF
            idx_cp(g, b).wait()
            store_cp(g - NBUF, b).wait()
            gather_cp(b).start()
            gather_cp(bp).wait()
            store_cp(g - 1, bp).start()
            idx_cp(g + NBUF - 1, bp).start()
        return carry

    lax.fori_loop(1, NROUND - 1, round_body, 0)

    g0 = (NROUND - 1) * NBUF
    for b in range(NBUF):
        g = g0 + b
        bp = (b - 1) % NBUF
        idx_cp(g, b).wait()
        store_cp(g - NBUF, b).wait()
        gather_cp(b).start()
        gather_cp(bp).wait()
        store_cp(g - 1, bp).start()
        if g + NBUF - 1 < NCHUNK:
            idx_cp(g + NBUF - 1, bp).start()
    gather_cp(NBUF - 1).wait()
    store_cp(NCHUNK - 1, NBUF - 1).start()
    for b in range(NBUF):
        store_cp(g0 + b, b).wait()


def kernel(X, table):
    flat_idx = X.reshape(B)
    out = _gather_kernel(flat_idx, table)
    return out.reshape(ROWS, COLS, DIM)


# speedup vs baseline: 1.0361x; 1.0361x over previous
import functools

import jax
import jax.numpy as jnp
from jax import lax
from jax.experimental import pallas as pl
from jax.experimental.pallas import tpu as pltpu
from jax.experimental.pallas import tpu_sc as plsc

VOCAB = 65024
DIM = 16
ROWS = 16384
COLS = 200
B = ROWS * COLS

NC, NS = 2, 16
NW = NC * NS
B_PER_W = B // NW
CHUNK = 400
NCHUNK = B_PER_W // CHUNK
NBUF = 8
GDEPTH = 4
NROUND = NCHUNK // NBUF
STAGE_ROWS = VOCAB // NS
SUBSTAGE = 508
NSTAGE = STAGE_ROWS // SUBSTAGE

_mesh = plsc.VectorSubcoreMesh(core_axis_name="c", subcore_axis_name="s")


@functools.partial(
    pl.kernel,
    out_type=jax.ShapeDtypeStruct((B, DIM), jnp.float32),
    mesh=_mesh,
    scratch_types=[pltpu.VMEM((CHUNK,), jnp.int32)] * NBUF
    + [pltpu.VMEM((CHUNK, DIM), jnp.float32)] * NBUF
    + [pltpu.SemaphoreType.DMA] * (3 * NBUF)
    + [
        pltpu.VMEM_SHARED((VOCAB, DIM), jnp.float32),
        pltpu.VMEM((SUBSTAGE, DIM), jnp.float32),
    ],
    compiler_params=pltpu.CompilerParams(use_tc_tiling_on_sc=False),
)
def _gather_kernel(x_hbm, table_hbm, out_hbm, *scratch):
    idx_v = scratch[0:NBUF]
    rows_v = scratch[NBUF : 2 * NBUF]
    sems = scratch[2 * NBUF : 5 * NBUF]
    sem_i = sems[0:NBUF]
    sem_g = sems[NBUF : 2 * NBUF]
    sem_o = sems[2 * NBUF : 3 * NBUF]
    tab_sh = scratch[5 * NBUF]
    stage_v = scratch[5 * NBUF + 1]

    sid = lax.axis_index("s")
    wid = sid * NC + lax.axis_index("c")
    base = wid * B_PER_W

    row0 = sid * STAGE_ROWS
    for k in range(NSTAGE):
        r0 = row0 + k * SUBSTAGE
        pltpu.sync_copy(table_hbm.at[pl.ds(r0, SUBSTAGE)], stage_v)
        pltpu.sync_copy(stage_v, tab_sh.at[pl.ds(r0, SUBSTAGE)])
    plsc.subcore_barrier()

    def idx_cp(g, b):
        return pltpu.make_async_copy(
            x_hbm.at[pl.ds(base + g * CHUNK, CHUNK)], idx_v[b], sem_i[b]
        )

    def gather_cp(b):
        return pltpu.make_async_copy(tab_sh.at[idx_v[b]], rows_v[b], sem_g[b])

    def store_cp(g, b):
        return pltpu.make_async_copy(
            rows_v[b], out_hbm.at[pl.ds(base + g * CHUNK, CHUNK)], sem_o[b]
        )

    def drain(g, b):
        gq = g - GDEPTH
        bq = gq % NBUF if isinstance(g, int) else (b - GDEPTH) % NBUF
        gather_cp(bq).wait()
        store_cp(gq, bq).start()
        return bq, gq

    for b in range(NBUF):
        idx_cp(b, b).start()
    for g in range(NBUF):
        idx_cp(g, g).wait()
        gather_cp(g).start()
        if g >= GDEPTH:
            bq, gq = drain(g, g)
            idx_cp(gq + NBUF, bq).start()

    def round_body(r, carry):
        g0 = r * NBUF
        for b in range(NBUF):
            g = g0 + b
            idx_cp(g, b).wait()
            store_cp(g - NBUF, b).wait()
            gather_cp(b).start()
            bq = (b - GDEPTH) % NBUF
            gather_cp(bq).wait()
            store_cp(g - GDEPTH, bq).start()
            idx_cp(g - GDEPTH + NBUF, bq).start()
        return carry

    lax.fori_loop(1, NROUND - 1, round_body, 0)

    g0 = (NROUND - 1) * NBUF
    for b in range(NBUF):
        g = g0 + b
        idx_cp(g, b).wait()
        store_cp(g - NBUF, b).wait()
        gather_cp(b).start()
        bq, gq = drain(g, b)
        if gq + NBUF < NCHUNK:
            idx_cp(gq + NBUF, bq).start()

    for g in range(NCHUNK, NCHUNK + GDEPTH):
        drain(g, g % NBUF)
    for b in range(NBUF):
        store_cp(NCHUNK - NBUF + b, (NCHUNK - NBUF + b) % NBUF).wait()


def kernel(X, table):
    flat_idx = X.reshape(B)
    out = _gather_kernel(flat_idx, table)
    return out.reshape(ROWS, COLS, DIM)
